# baseline (speedup 1.0000x reference)
import functools

import jax
import jax.numpy as jnp
import numpy as np
from jax import lax
from jax.experimental import pallas as pl
from jax.experimental.pallas import tpu as pltpu
from jax.experimental.pallas import tpu_sc as plsc


def _ln(h, g, b):
    mu = jnp.mean(h, axis=-1, keepdims=True)
    var = jnp.mean((h - mu) ** 2, axis=-1, keepdims=True)
    return (h - mu) / jnp.sqrt(var + 1e-5) * g + b


def _dot(a, b):
    return jnp.dot(a, b, preferred_element_type=jnp.float32)


def _full(shape):
    return pl.BlockSpec(shape, lambda i: (0,) * len(shape))


def _make_tc_encode(n, d, r):
    def body(x_ref, encW_ref, encb_ref, encg_ref, encbb_ref, g0_ref, b0_ref,
             W1_ref, b1_ref, xcur_ref, hh_ref, T_ref):
        h = _dot(x_ref[...], encW_ref[...]) + encb_ref[...]
        h = _ln(h, encg_ref[...], encbb_ref[...])
        xcur_ref[...] = h
        hh = jnp.maximum(_ln(h, g0_ref[...], b0_ref[...]), 0.0)
        hh_ref[...] = hh
        W1 = W1_ref[...]
        A = _dot(hh, W1[:d, :]) + b1_ref[...]
        B = _dot(hh, W1[d:, :])
        T_ref[0] = A[:, :d]
        T_ref[1] = A[:, d:]
        T_ref[2] = B[:, :d]
        T_ref[3] = B[:, d:]

    row = pl.BlockSpec((r, d), lambda i: (i, 0))
    return pl.pallas_call(
        body,
        grid=(n // r,),
        in_specs=[row, _full((d, d)), _full((1, d)), _full((1, d)),
                  _full((1, d)), _full((1, d)), _full((1, d)),
                  _full((2 * d, 2 * d)), _full((1, 2 * d))],
        out_specs=[row, row, pl.BlockSpec((4, r, d), lambda i: (0, i, 0))],
        out_shape=[jax.ShapeDtypeStruct((n, d), jnp.float32),
                   jax.ShapeDtypeStruct((n, d), jnp.float32),
                   jax.ShapeDtypeStruct((4, n, d), jnp.float32)],
    )


def _make_tc_update(n, d, r, has_next):
    def body(xcur_ref, hh_ref, S0_ref, S1_ref, cnt0_ref, cnt1_ref, W2_ref,
             b2_ref, Wr_ref, *rest):
        if has_next:
            gn_ref, bn_ref, W1n_ref, b1n_ref = rest[:4]
            xnew_ref, hh2_ref, T_ref = rest[4:]
        else:
            (xnew_ref,) = rest
        cc = cnt0_ref[...][:, 0:1] + cnt1_ref[...][:, 0:1]
        W2 = W2_ref[...]
        m = _dot(S0_ref[...], W2[:d, :]) + _dot(S1_ref[...], W2[d:, :])
        mean = m / jnp.maximum(cc, 1.0) + jnp.minimum(cc, 1.0) * b2_ref[...]
        out = mean + _dot(hh_ref[...], Wr_ref[...])
        xnew = xcur_ref[...] + out
        xnew_ref[...] = xnew
        if has_next:
            hh2 = jnp.maximum(_ln(xnew, gn_ref[...], bn_ref[...]), 0.0)
            hh2_ref[...] = hh2
            W1 = W1n_ref[...]
            A = _dot(hh2, W1[:d, :]) + b1n_ref[...]
            B = _dot(hh2, W1[d:, :])
            T_ref[0] = A[:, :d]
            T_ref[1] = A[:, d:]
            T_ref[2] = B[:, :d]
            T_ref[3] = B[:, d:]

    row = pl.BlockSpec((r, d), lambda i: (i, 0))
    in_specs = [row, row, row, row, row, row,
                _full((2 * d, d)), _full((1, d)), _full((d, d))]
    out_specs = [row]
    out_shape = [jax.ShapeDtypeStruct((n, d), jnp.float32)]
    if has_next:
        in_specs += [_full((1, d)), _full((1, d)), _full((2 * d, 2 * d)),
                     _full((1, 2 * d))]
        out_specs += [row, pl.BlockSpec((4, r, d), lambda i: (0, i, 0))]
        out_shape += [jax.ShapeDtypeStruct((n, d), jnp.float32),
                      jax.ShapeDtypeStruct((4, n, d), jnp.float32)]
    return pl.pallas_call(body, grid=(n // r,), in_specs=in_specs,
                          out_specs=out_specs, out_shape=out_shape)


def _sc_mesh_params(n, e):
    info = plsc.get_sparse_core_info()
    ns = info.num_subcores
    CH = 80
    per_tile = e // ns
    nch = per_tile // CH
    assert per_tile % CH == 0 and nch % 2 == 0
    rows_main = (n // ns) // 8 * 8
    rows_tail = n - rows_main * ns
    assert rows_tail % 8 == 0
    return ns, CH, per_tile, nch, rows_main, rows_tail


def _make_sc_segment(n, d, e):
    ns, CH0, per_tile, _, rows_main, rows_tail = _sc_mesh_params(n, e)
    NB = 5
    CH = 32
    nch = per_tile // CH
    assert per_tile % CH == 0 and nch % NB == 0

    mesh = plsc.VectorSubcoreMesh(core_axis_name="c", subcore_axis_name="s")
    out_type = (jax.ShapeDtypeStruct((n, d), jnp.float32),
                jax.ShapeDtypeStruct((n, d), jnp.float32))
    idx = lambda: pltpu.VMEM((CH,), jnp.int32)
    row = lambda: pltpu.VMEM((CH, d), jnp.float32)
    scratch = tuple(
        kind() for _ in range(NB) for kind in (idx, idx, idx, idx, idx,
                                               row, row)
    ) + (pltpu.VMEM_SHARED((n, d), jnp.float32),) \
      + (pltpu.SemaphoreType.DMA,) * (5 * NB)

    def body(T_ref, ed_ref, S0_ref, S1_ref, *rest):
        nscr = 7 * NB
        bufs_flat = rest[:nscr]
        S_sh = rest[nscr]
        sems = rest[nscr + 1:]
        B = []
        for b in range(NB):
            di, si, gA, gB, dc, a, bv = bufs_flat[7 * b:7 * b + 7]
            sI, sJ, sA, sB_, sS = sems[5 * b:5 * b + 5]
            B.append({"di": di, "si": si, "gA": gA, "gB": gB, "dc": dc,
                      "a": a, "b": bv, "sI": sI, "sJ": sJ, "sA": sA,
                      "sB": sB_, "sS": sS})
        c = lax.axis_index("c")
        s = lax.axis_index("s")
        base = s * per_tile
        offA = c * n
        offB = (2 + c) * n
        a0 = B[0]["a"]

        @pl.loop(0, CH)
        def _(rr):
            for j in range(d // 16):
                a0[rr, pl.ds(j * 16, 16)] = jnp.zeros((16,), jnp.float32)

        nz = rows_main // CH
        rem = rows_main - nz * CH
        for k in range(nz):
            rs = pl.ds(s * rows_main + k * CH, CH)
            pltpu.sync_copy(a0, S_sh.at[rs])
        if rem:
            rs = pl.ds(s * rows_main + nz * CH, rem)
            pltpu.sync_copy(a0.at[pl.ds(0, rem)], S_sh.at[rs])
        if rows_tail:
            @pl.when(s == 0)
            def _():
                rt = pl.ds(rows_main * ns, rows_tail)
                pltpu.sync_copy(a0.at[pl.ds(0, rows_tail)], S_sh.at[rt])
        plsc.subcore_barrier()

        def issue_idx(q, u):
            off = base + q * CH
            pltpu.async_copy(ed_ref.at[pl.ds(off, CH)], u["di"], u["sI"])
            pltpu.async_copy(ed_ref.at[pl.ds(e + off, CH)], u["si"], u["sJ"])

        def wait_idx(u):
            pltpu.make_async_copy(ed_ref.at[pl.ds(0, CH)], u["di"],
                                  u["sI"]).wait()
            pltpu.make_async_copy(ed_ref.at[pl.ds(0, CH)], u["si"],
                                  u["sJ"]).wait()

        offs16 = list(range(0, CH - 16, 16)) + [CH - 16]

        def adjust_and_gather(u):
            for o in offs16:
                sl = pl.ds(o, 16)
                u["gA"][sl] = u["di"][sl] + offA
                u["gB"][sl] = u["si"][sl] + offB
            pltpu.async_copy(T_ref.at[u["gA"]], u["a"], u["sA"])
            pltpu.async_copy(T_ref.at[u["gB"]], u["b"], u["sB"])

        def wait_gather(u):
            pltpu.make_async_copy(T_ref.at[u["gA"]], u["a"], u["sA"]).wait()
            pltpu.make_async_copy(T_ref.at[u["gB"]], u["b"], u["sB"]).wait()

        def compute(u):
            for o in offs16:
                sl = pl.ds(o, 16)
                u["dc"][sl] = u["gA"][sl] - offA
            aV, bV = u["a"], u["b"]

            @pl.loop(0, CH, unroll=4)
            def _(rr):
                for j in range(d // 16):
                    sl = pl.ds(j * 16, 16)
                    aV[rr, sl] = jnp.maximum(aV[rr, sl] + bV[rr, sl], 0.0)

        def issue_scatter(u):
            pltpu.async_copy(u["a"], S_sh.at[u["dc"]], u["sS"], add=True)

        def drain_scatter(u):
            pltpu.make_async_copy(u["a"], S_sh.at[u["dc"]], u["sS"]).wait()

        for b in range(NB):
            issue_idx(b, B[b])
            wait_idx(B[b])
            adjust_and_gather(B[b])
            issue_idx(NB + b, B[b])

        @pl.loop(0, nch // NB)
        def _(i):
            for b in range(NB):
                u = B[b]
                wait_gather(u)
                compute(u)
                issue_scatter(u)
                pb = B[(b - 1) % NB]
                qq = i * NB + b - 1

                @pl.when((qq >= 0) & (qq + NB < nch))
                def _():
                    wait_idx(pb)
                    drain_scatter(pb)
                    adjust_and_gather(pb)

                    @pl.when(qq + 2 * NB < nch)
                    def _():
                        issue_idx(qq + 2 * NB, pb)

        for b in range(NB):
            drain_scatter(B[b])
        plsc.subcore_barrier()

        def copy_rows(src, dst):
            rs = pl.ds(s * rows_main, rows_main)
            pltpu.sync_copy(src.at[rs], dst.at[rs])
            if rows_tail:
                @pl.when(s == 0)
                def _():
                    rt = pl.ds(rows_main * ns, rows_tail)
                    pltpu.sync_copy(src.at[rt], dst.at[rt])

        @pl.when(c == 0)
        def _():
            copy_rows(S_sh, S0_ref)

        @pl.when(c == 1)
        def _():
            copy_rows(S_sh, S1_ref)

    return pl.kernel(body, out_type=out_type, mesh=mesh,
                     scratch_types=scratch)


def _make_sc_count(n, d, e):
    ns, CH, per_tile, nch, rows_main, rows_tail = _sc_mesh_params(n, e)
    per_w = e // (2 * ns)
    nchw = per_w // CH
    assert per_w % CH == 0
    pair = nchw // 2
    mesh = plsc.VectorSubcoreMesh(core_axis_name="c", subcore_axis_name="s")
    idx = lambda: pltpu.VMEM((CH,), jnp.int32)

    def body(ed_ref, ones_ref, cnt0_ref, cnt1_ref,
             di0, di1, dc0, dc1, ones_v, zv, cnt_sh, sI0, sI1, sS0, sS1):
        c = lax.axis_index("c")
        s = lax.axis_index("s")
        base = (c * ns + s) * per_w
        B = ({"di": di0, "dc": dc0, "sI": sI0, "sS": sS0},
             {"di": di1, "dc": dc1, "sI": sI1, "sS": sS1})

        def copy_rows(src, dst):
            rs = pl.ds(s * rows_main, rows_main)
            pltpu.sync_copy(src.at[rs], dst.at[rs])
            if rows_tail:
                @pl.when(s == 0)
                def _():
                    rt = pl.ds(rows_main * ns, rows_tail)
                    pltpu.sync_copy(src.at[rt], dst.at[rt])

        @pl.loop(0, CH)
        def _(rr):
            for j in range(d // 16):
                zv[rr, pl.ds(j * 16, 16)] = jnp.zeros((16,), jnp.float32)

        nz = rows_main // CH
        rem = rows_main - nz * CH
        for k in range(nz):
            rs = pl.ds(s * rows_main + k * CH, CH)
            pltpu.sync_copy(zv, cnt_sh.at[rs])
        if rem:
            rs = pl.ds(s * rows_main + nz * CH, rem)
            pltpu.sync_copy(zv.at[pl.ds(0, rem)], cnt_sh.at[rs])
        if rows_tail:
            @pl.when(s == 0)
            def _():
                rt = pl.ds(rows_main * ns, rows_tail)
                pltpu.sync_copy(zv.at[pl.ds(0, rows_tail)], cnt_sh.at[rt])
        pltpu.sync_copy(ones_ref, ones_v)
        plsc.subcore_barrier()

        def issue_idx(q, u):
            pltpu.async_copy(ed_ref.at[pl.ds(base + q * CH, CH)],
                             u["di"], u["sI"])

        def wait_idx(u):
            pltpu.make_async_copy(ed_ref.at[pl.ds(0, CH)], u["di"],
                                  u["sI"]).wait()

        issue_idx(0, B[0])
        issue_idx(1, B[1])

        @pl.loop(0, pair)
        def _(i):
            q = i * 2
            for b in range(2):
                u = B[b]
                wait_idx(u)

                @pl.when(i > 0)
                def _():
                    pltpu.make_async_copy(ones_v, cnt_sh.at[u["dc"]],
                                          u["sS"]).wait()

                for j in range(CH // 16):
                    sl = pl.ds(j * 16, 16)
                    u["dc"][sl] = u["di"][sl]

                @pl.when(q + 2 + b < nchw)
                def _():
                    issue_idx(q + 2 + b, u)

                pltpu.async_copy(ones_v, cnt_sh.at[u["dc"]], u["sS"],
                                 add=True)

        if nchw % 2:
            u = B[0]
            wait_idx(u)
            pltpu.make_async_copy(ones_v, cnt_sh.at[u["dc"]], u["sS"]).wait()
            for j in range(CH // 16):
                sl = pl.ds(j * 16, 16)
                u["dc"][sl] = u["di"][sl]
            pltpu.async_copy(ones_v, cnt_sh.at[u["dc"]], u["sS"], add=True)

        pltpu.make_async_copy(ones_v, cnt_sh.at[B[0]["dc"]], B[0]["sS"]).wait()
        pltpu.make_async_copy(ones_v, cnt_sh.at[B[1]["dc"]], B[1]["sS"]).wait()
        plsc.subcore_barrier()

        @pl.when(c == 0)
        def _():
            copy_rows(cnt_sh, cnt0_ref)

        @pl.when(c == 1)
        def _():
            copy_rows(cnt_sh, cnt1_ref)

    return pl.kernel(
        body,
        out_type=(jax.ShapeDtypeStruct((n, d), jnp.float32),
                  jax.ShapeDtypeStruct((n, d), jnp.float32)),
        mesh=mesh,
        scratch_types=(idx(), idx(), idx(), idx(),
                       pltpu.VMEM((CH, d), jnp.float32),
                       pltpu.VMEM((CH, d), jnp.float32),
                       pltpu.VMEM_SHARED((n, d), jnp.float32))
                      + (pltpu.SemaphoreType.DMA,) * 4)


def kernel(x, edge_index, enc_W, enc_b, enc_ln_g, enc_ln_b, ln_g, ln_b,
           W1, b1, W2, b2, Wr):
    n, d = x.shape
    e = edge_index.shape[1]
    nlayers = ln_g.shape[0]
    r = 1000

    src = edge_index[0].astype(jnp.int32)
    dst = edge_index[1].astype(jnp.int32)
    edges2 = jnp.concatenate([dst, src])
    row2 = lambda v: v.reshape(1, -1).astype(jnp.float32)

    encode = _make_tc_encode(n, d, r)
    xcur, hh, T = encode(x, enc_W, row2(enc_b), row2(enc_ln_g),
                         row2(enc_ln_b), row2(ln_g[0]), row2(ln_b[0]),
                         W1[0], row2(b1[0]))

    sc_seg = _make_sc_segment(n, d, e)
    ones_rows = jnp.ones((80, d), jnp.float32)
    cnt0, cnt1 = _make_sc_count(n, d, e)(edges2, ones_rows)

    for l in range(nlayers):
        S0, S1 = sc_seg(T.reshape(4 * n, d), edges2)
        W2p = W2[l]
        has_next = l + 1 < nlayers
        update = _make_tc_update(n, d, r, has_next)
        if has_next:
            xcur, hh, T = update(xcur, hh, S0, S1, cnt0, cnt1, W2p,
                                 row2(b2[l]), Wr[l], row2(ln_g[l + 1]),
                                 row2(ln_b[l + 1]), W1[l + 1],
                                 row2(b1[l + 1]))
        else:
            (xcur,) = update(xcur, hh, S0, S1, cnt0, cnt1, W2p,
                             row2(b2[l]), Wr[l])
    return xcur


# speedup vs baseline: 1.0043x; 1.0043x over previous
import functools

import jax
import jax.numpy as jnp
import numpy as np
from jax import lax
from jax.experimental import pallas as pl
from jax.experimental.pallas import tpu as pltpu
from jax.experimental.pallas import tpu_sc as plsc


def _ln(h, g, b):
    mu = jnp.mean(h, axis=-1, keepdims=True)
    var = jnp.mean((h - mu) ** 2, axis=-1, keepdims=True)
    return (h - mu) / jnp.sqrt(var + 1e-5) * g + b


def _dot(a, b):
    return jnp.dot(a, b, preferred_element_type=jnp.float32)


def _full(shape):
    return pl.BlockSpec(shape, lambda i: (0,) * len(shape))


def _make_tc_encode(n, d, r):
    def body(x_ref, encW_ref, encb_ref, encg_ref, encbb_ref, g0_ref, b0_ref,
             W1_ref, b1_ref, xcur_ref, hh_ref, T_ref):
        h = _dot(x_ref[...], encW_ref[...]) + encb_ref[...]
        h = _ln(h, encg_ref[...], encbb_ref[...])
        xcur_ref[...] = h
        hh = jnp.maximum(_ln(h, g0_ref[...], b0_ref[...]), 0.0)
        hh_ref[...] = hh
        W1 = W1_ref[...]
        A = _dot(hh, W1[:d, :]) + b1_ref[...]
        B = _dot(hh, W1[d:, :])
        T_ref[0] = A[:, :d]
        T_ref[1] = A[:, d:]
        T_ref[2] = B[:, :d]
        T_ref[3] = B[:, d:]

    row = pl.BlockSpec((r, d), lambda i: (i, 0))
    return pl.pallas_call(
        body,
        grid=(n // r,),
        in_specs=[row, _full((d, d)), _full((1, d)), _full((1, d)),
                  _full((1, d)), _full((1, d)), _full((1, d)),
                  _full((2 * d, 2 * d)), _full((1, 2 * d))],
        out_specs=[row, row, pl.BlockSpec((4, r, d), lambda i: (0, i, 0))],
        out_shape=[jax.ShapeDtypeStruct((n, d), jnp.float32),
                   jax.ShapeDtypeStruct((n, d), jnp.float32),
                   jax.ShapeDtypeStruct((4, n, d), jnp.float32)],
    )


def _make_tc_update(n, d, r, has_next):
    def body(xcur_ref, hh_ref, S0_ref, S1_ref, cnt0_ref, cnt1_ref, W2_ref,
             b2_ref, Wr_ref, *rest):
        if has_next:
            gn_ref, bn_ref, W1n_ref, b1n_ref = rest[:4]
            xnew_ref, hh2_ref, T_ref = rest[4:]
        else:
            (xnew_ref,) = rest
        cc = cnt0_ref[...][:, 0:1] + cnt1_ref[...][:, 0:1]
        W2 = W2_ref[...]
        m = _dot(S0_ref[...], W2[:d, :]) + _dot(S1_ref[...], W2[d:, :])
        mean = m / jnp.maximum(cc, 1.0) + jnp.minimum(cc, 1.0) * b2_ref[...]
        out = mean + _dot(hh_ref[...], Wr_ref[...])
        xnew = xcur_ref[...] + out
        xnew_ref[...] = xnew
        if has_next:
            hh2 = jnp.maximum(_ln(xnew, gn_ref[...], bn_ref[...]), 0.0)
            hh2_ref[...] = hh2
            W1 = W1n_ref[...]
            A = _dot(hh2, W1[:d, :]) + b1n_ref[...]
            B = _dot(hh2, W1[d:, :])
            T_ref[0] = A[:, :d]
            T_ref[1] = A[:, d:]
            T_ref[2] = B[:, :d]
            T_ref[3] = B[:, d:]

    row = pl.BlockSpec((r, d), lambda i: (i, 0))
    in_specs = [row, row, row, row, row, row,
                _full((2 * d, d)), _full((1, d)), _full((d, d))]
    out_specs = [row]
    out_shape = [jax.ShapeDtypeStruct((n, d), jnp.float32)]
    if has_next:
        in_specs += [_full((1, d)), _full((1, d)), _full((2 * d, 2 * d)),
                     _full((1, 2 * d))]
        out_specs += [row, pl.BlockSpec((4, r, d), lambda i: (0, i, 0))]
        out_shape += [jax.ShapeDtypeStruct((n, d), jnp.float32),
                      jax.ShapeDtypeStruct((4, n, d), jnp.float32)]
    return pl.pallas_call(body, grid=(n // r,), in_specs=in_specs,
                          out_specs=out_specs, out_shape=out_shape)


def _sc_mesh_params(n, e):
    info = plsc.get_sparse_core_info()
    ns = info.num_subcores
    CH = 80
    per_tile = e // ns
    nch = per_tile // CH
    assert per_tile % CH == 0 and nch % 2 == 0
    rows_main = (n // ns) // 8 * 8
    rows_tail = n - rows_main * ns
    assert rows_tail % 8 == 0
    return ns, CH, per_tile, nch, rows_main, rows_tail


def _make_sc_segment(n, d, e):
    ns, CH0, per_tile, _, rows_main, rows_tail = _sc_mesh_params(n, e)
    NB = 4
    CH = 40
    nch = per_tile // CH
    assert per_tile % CH == 0 and nch % NB == 0

    mesh = plsc.VectorSubcoreMesh(core_axis_name="c", subcore_axis_name="s")
    out_type = (jax.ShapeDtypeStruct((n, d), jnp.float32),
                jax.ShapeDtypeStruct((n, d), jnp.float32))
    idx = lambda: pltpu.VMEM((CH,), jnp.int32)
    row = lambda: pltpu.VMEM((CH, d), jnp.float32)
    scratch = tuple(
        kind() for _ in range(NB) for kind in (idx, idx, idx, idx, idx,
                                               row, row)
    ) + (pltpu.VMEM_SHARED((n, d), jnp.float32),) \
      + (pltpu.SemaphoreType.DMA,) * (5 * NB)

    def body(T_ref, ed_ref, S0_ref, S1_ref, *rest):
        nscr = 7 * NB
        bufs_flat = rest[:nscr]
        S_sh = rest[nscr]
        sems = rest[nscr + 1:]
        B = []
        for b in range(NB):
            di, si, gA, gB, dc, a, bv = bufs_flat[7 * b:7 * b + 7]
            sI, sJ, sA, sB_, sS = sems[5 * b:5 * b + 5]
            B.append({"di": di, "si": si, "gA": gA, "gB": gB, "dc": dc,
                      "a": a, "b": bv, "sI": sI, "sJ": sJ, "sA": sA,
                      "sB": sB_, "sS": sS})
        c = lax.axis_index("c")
        s = lax.axis_index("s")
        base = s * per_tile
        offA = c * n
        offB = (2 + c) * n
        a0 = B[0]["a"]

        @pl.loop(0, CH)
        def _(rr):
            for j in range(d // 16):
                a0[rr, pl.ds(j * 16, 16)] = jnp.zeros((16,), jnp.float32)

        nz = rows_main // CH
        rem = rows_main - nz * CH
        for k in range(nz):
            rs = pl.ds(s * rows_main + k * CH, CH)
            pltpu.sync_copy(a0, S_sh.at[rs])
        if rem:
            rs = pl.ds(s * rows_main + nz * CH, rem)
            pltpu.sync_copy(a0.at[pl.ds(0, rem)], S_sh.at[rs])
        if rows_tail:
            @pl.when(s == 0)
            def _():
                rt = pl.ds(rows_main * ns, rows_tail)
                pltpu.sync_copy(a0.at[pl.ds(0, rows_tail)], S_sh.at[rt])
        plsc.subcore_barrier()

        def issue_idx(q, u):
            off = base + q * CH
            pltpu.async_copy(ed_ref.at[pl.ds(off, CH)], u["di"], u["sI"])
            pltpu.async_copy(ed_ref.at[pl.ds(e + off, CH)], u["si"], u["sJ"])

        def wait_idx(u):
            pltpu.make_async_copy(ed_ref.at[pl.ds(0, CH)], u["di"],
                                  u["sI"]).wait()
            pltpu.make_async_copy(ed_ref.at[pl.ds(0, CH)], u["si"],
                                  u["sJ"]).wait()

        offs16 = list(range(0, CH - 16, 16)) + [CH - 16]

        def adjust_and_gather(u):
            for o in offs16:
                sl = pl.ds(o, 16)
                u["gA"][sl] = u["di"][sl] + offA
                u["gB"][sl] = u["si"][sl] + offB
            pltpu.async_copy(T_ref.at[u["gA"]], u["a"], u["sA"])
            pltpu.async_copy(T_ref.at[u["gB"]], u["b"], u["sB"])

        def wait_gather(u):
            pltpu.make_async_copy(T_ref.at[u["gA"]], u["a"], u["sA"]).wait()
            pltpu.make_async_copy(T_ref.at[u["gB"]], u["b"], u["sB"]).wait()

        def compute(u):
            for o in offs16:
                sl = pl.ds(o, 16)
                u["dc"][sl] = u["gA"][sl] - offA
            aV, bV = u["a"], u["b"]

            @pl.loop(0, CH, unroll=4)
            def _(rr):
                for j in range(d // 16):
                    sl = pl.ds(j * 16, 16)
                    aV[rr, sl] = jnp.maximum(aV[rr, sl] + bV[rr, sl], 0.0)

        def issue_scatter(u):
            pltpu.async_copy(u["a"], S_sh.at[u["dc"]], u["sS"], add=True)

        def drain_scatter(u):
            pltpu.make_async_copy(u["a"], S_sh.at[u["dc"]], u["sS"]).wait()

        for b in range(NB):
            issue_idx(b, B[b])
            wait_idx(B[b])
            adjust_and_gather(B[b])
            issue_idx(NB + b, B[b])

        @pl.loop(0, nch // NB)
        def _(i):
            for b in range(NB):
                u = B[b]
                wait_gather(u)
                compute(u)
                issue_scatter(u)
                pb = B[(b - 1) % NB]
                qq = i * NB + b - 1

                @pl.when((qq >= 0) & (qq + NB < nch))
                def _():
                    wait_idx(pb)
                    drain_scatter(pb)
                    adjust_and_gather(pb)

                    @pl.when(qq + 2 * NB < nch)
                    def _():
                        issue_idx(qq + 2 * NB, pb)

        for b in range(NB):
            drain_scatter(B[b])
        plsc.subcore_barrier()

        def copy_rows(src, dst):
            rs = pl.ds(s * rows_main, rows_main)
            pltpu.sync_copy(src.at[rs], dst.at[rs])
            if rows_tail:
                @pl.when(s == 0)
                def _():
                    rt = pl.ds(rows_main * ns, rows_tail)
                    pltpu.sync_copy(src.at[rt], dst.at[rt])

        @pl.when(c == 0)
        def _():
            copy_rows(S_sh, S0_ref)

        @pl.when(c == 1)
        def _():
            copy_rows(S_sh, S1_ref)

    return pl.kernel(body, out_type=out_type, mesh=mesh,
                     scratch_types=scratch)


def _make_sc_count(n, d, e):
    ns, CH, per_tile, nch, rows_main, rows_tail = _sc_mesh_params(n, e)
    per_w = e // (2 * ns)
    nchw = per_w // CH
    assert per_w % CH == 0
    pair = nchw // 2
    mesh = plsc.VectorSubcoreMesh(core_axis_name="c", subcore_axis_name="s")
    idx = lambda: pltpu.VMEM((CH,), jnp.int32)

    def body(ed_ref, ones_ref, cnt0_ref, cnt1_ref,
             di0, di1, dc0, dc1, ones_v, zv, cnt_sh, sI0, sI1, sS0, sS1):
        c = lax.axis_index("c")
        s = lax.axis_index("s")
        base = (c * ns + s) * per_w
        B = ({"di": di0, "dc": dc0, "sI": sI0, "sS": sS0},
             {"di": di1, "dc": dc1, "sI": sI1, "sS": sS1})

        def copy_rows(src, dst):
            rs = pl.ds(s * rows_main, rows_main)
            pltpu.sync_copy(src.at[rs], dst.at[rs])
            if rows_tail:
                @pl.when(s == 0)
                def _():
                    rt = pl.ds(rows_main * ns, rows_tail)
                    pltpu.sync_copy(src.at[rt], dst.at[rt])

        @pl.loop(0, CH)
        def _(rr):
            for j in range(d // 16):
                zv[rr, pl.ds(j * 16, 16)] = jnp.zeros((16,), jnp.float32)

        nz = rows_main // CH
        rem = rows_main - nz * CH
        for k in range(nz):
            rs = pl.ds(s * rows_main + k * CH, CH)
            pltpu.sync_copy(zv, cnt_sh.at[rs])
        if rem:
            rs = pl.ds(s * rows_main + nz * CH, rem)
            pltpu.sync_copy(zv.at[pl.ds(0, rem)], cnt_sh.at[rs])
        if rows_tail:
            @pl.when(s == 0)
            def _():
                rt = pl.ds(rows_main * ns, rows_tail)
                pltpu.sync_copy(zv.at[pl.ds(0, rows_tail)], cnt_sh.at[rt])
        pltpu.sync_copy(ones_ref, ones_v)
        plsc.subcore_barrier()

        def issue_idx(q, u):
            pltpu.async_copy(ed_ref.at[pl.ds(base + q * CH, CH)],
                             u["di"], u["sI"])

        def wait_idx(u):
            pltpu.make_async_copy(ed_ref.at[pl.ds(0, CH)], u["di"],
                                  u["sI"]).wait()

        issue_idx(0, B[0])
        issue_idx(1, B[1])

        @pl.loop(0, pair)
        def _(i):
            q = i * 2
            for b in range(2):
                u = B[b]
                wait_idx(u)

                @pl.when(i > 0)
                def _():
                    pltpu.make_async_copy(ones_v, cnt_sh.at[u["dc"]],
                                          u["sS"]).wait()

                for j in range(CH // 16):
                    sl = pl.ds(j * 16, 16)
                    u["dc"][sl] = u["di"][sl]

                @pl.when(q + 2 + b < nchw)
                def _():
                    issue_idx(q + 2 + b, u)

                pltpu.async_copy(ones_v, cnt_sh.at[u["dc"]], u["sS"],
                                 add=True)

        if nchw % 2:
            u = B[0]
            wait_idx(u)
            pltpu.make_async_copy(ones_v, cnt_sh.at[u["dc"]], u["sS"]).wait()
            for j in range(CH // 16):
                sl = pl.ds(j * 16, 16)
                u["dc"][sl] = u["di"][sl]
            pltpu.async_copy(ones_v, cnt_sh.at[u["dc"]], u["sS"], add=True)

        pltpu.make_async_copy(ones_v, cnt_sh.at[B[0]["dc"]], B[0]["sS"]).wait()
        pltpu.make_async_copy(ones_v, cnt_sh.at[B[1]["dc"]], B[1]["sS"]).wait()
        plsc.subcore_barrier()

        @pl.when(c == 0)
        def _():
            copy_rows(cnt_sh, cnt0_ref)

        @pl.when(c == 1)
        def _():
            copy_rows(cnt_sh, cnt1_ref)

    return pl.kernel(
        body,
        out_type=(jax.ShapeDtypeStruct((n, d), jnp.float32),
                  jax.ShapeDtypeStruct((n, d), jnp.float32)),
        mesh=mesh,
        scratch_types=(idx(), idx(), idx(), idx(),
                       pltpu.VMEM((CH, d), jnp.float32),
                       pltpu.VMEM((CH, d), jnp.float32),
                       pltpu.VMEM_SHARED((n, d), jnp.float32))
                      + (pltpu.SemaphoreType.DMA,) * 4)


def kernel(x, edge_index, enc_W, enc_b, enc_ln_g, enc_ln_b, ln_g, ln_b,
           W1, b1, W2, b2, Wr):
    n, d = x.shape
    e = edge_index.shape[1]
    nlayers = ln_g.shape[0]
    r = 1000

    src = edge_index[0].astype(jnp.int32)
    dst = edge_index[1].astype(jnp.int32)
    edges2 = jnp.concatenate([dst, src])
    row2 = lambda v: v.reshape(1, -1).astype(jnp.float32)

    encode = _make_tc_encode(n, d, r)
    xcur, hh, T = encode(x, enc_W, row2(enc_b), row2(enc_ln_g),
                         row2(enc_ln_b), row2(ln_g[0]), row2(ln_b[0]),
                         W1[0], row2(b1[0]))

    sc_seg = _make_sc_segment(n, d, e)
    ones_rows = jnp.ones((80, d), jnp.float32)
    cnt0, cnt1 = _make_sc_count(n, d, e)(edges2, ones_rows)

    for l in range(nlayers):
        S0, S1 = sc_seg(T.reshape(4 * n, d), edges2)
        W2p = W2[l]
        has_next = l + 1 < nlayers
        update = _make_tc_update(n, d, r, has_next)
        if has_next:
            xcur, hh, T = update(xcur, hh, S0, S1, cnt0, cnt1, W2p,
                                 row2(b2[l]), Wr[l], row2(ln_g[l + 1]),
                                 row2(ln_b[l + 1]), W1[l + 1],
                                 row2(b1[l + 1]))
        else:
            (xcur,) = update(xcur, hh, S0, S1, cnt0, cnt1, W2p,
                             row2(b2[l]), Wr[l])
    return xcur


# speedup vs baseline: 2.4908x; 2.4802x over previous
import functools

import jax
import jax.numpy as jnp
import numpy as np
from jax import lax
from jax.experimental import pallas as pl
from jax.experimental.pallas import tpu as pltpu
from jax.experimental.pallas import tpu_sc as plsc


def _ln(h, g, b):
    mu = jnp.mean(h, axis=-1, keepdims=True)
    var = jnp.mean((h - mu) ** 2, axis=-1, keepdims=True)
    return (h - mu) / jnp.sqrt(var + 1e-5) * g + b


def _dot(a, b):
    return jnp.dot(a, b, preferred_element_type=jnp.float32)


def _full(shape):
    return pl.BlockSpec(shape, lambda i: (0,) * len(shape))


def _make_tc_encode(n, d, r):
    def body(x_ref, encW_ref, encb_ref, encg_ref, encbb_ref, g0_ref, b0_ref,
             W1_ref, b1_ref, xcur_ref, hh_ref, T_ref):
        h = _dot(x_ref[...], encW_ref[...]) + encb_ref[...]
        h = _ln(h, encg_ref[...], encbb_ref[...])
        xcur_ref[...] = h
        hh = jnp.maximum(_ln(h, g0_ref[...], b0_ref[...]), 0.0)
        hh_ref[...] = hh
        W1 = W1_ref[...]
        A = _dot(hh, W1[:d, :]) + b1_ref[...]
        B = _dot(hh, W1[d:, :])
        T_ref[0] = A[:, :d]
        T_ref[1] = A[:, d:]
        T_ref[2] = B[:, :d]
        T_ref[3] = B[:, d:]

    row = pl.BlockSpec((r, d), lambda i: (i, 0))
    return pl.pallas_call(
        body,
        grid=(n // r,),
        in_specs=[row, _full((d, d)), _full((1, d)), _full((1, d)),
                  _full((1, d)), _full((1, d)), _full((1, d)),
                  _full((2 * d, 2 * d)), _full((1, 2 * d))],
        out_specs=[row, row, pl.BlockSpec((4, r, d), lambda i: (0, i, 0))],
        out_shape=[jax.ShapeDtypeStruct((n, d), jnp.float32),
                   jax.ShapeDtypeStruct((n, d), jnp.float32),
                   jax.ShapeDtypeStruct((4, n, d), jnp.float32)],
    )


def _make_tc_update(n, d, r, has_next):
    def body(xcur_ref, hh_ref, S0_ref, S1_ref, cnt0_ref, cnt1_ref, W2_ref,
             b2_ref, Wr_ref, *rest):
        if has_next:
            gn_ref, bn_ref, W1n_ref, b1n_ref = rest[:4]
            xnew_ref, hh2_ref, T_ref = rest[4:]
        else:
            (xnew_ref,) = rest
        cc = cnt0_ref[...][:, 0:1] + cnt1_ref[...][:, 0:1]
        W2 = W2_ref[...]
        m = _dot(S0_ref[...], W2[:d, :]) + _dot(S1_ref[...], W2[d:, :])
        mean = m / jnp.maximum(cc, 1.0) + jnp.minimum(cc, 1.0) * b2_ref[...]
        out = mean + _dot(hh_ref[...], Wr_ref[...])
        xnew = xcur_ref[...] + out
        xnew_ref[...] = xnew
        if has_next:
            hh2 = jnp.maximum(_ln(xnew, gn_ref[...], bn_ref[...]), 0.0)
            hh2_ref[...] = hh2
            W1 = W1n_ref[...]
            A = _dot(hh2, W1[:d, :]) + b1n_ref[...]
            B = _dot(hh2, W1[d:, :])
            T_ref[0] = A[:, :d]
            T_ref[1] = A[:, d:]
            T_ref[2] = B[:, :d]
            T_ref[3] = B[:, d:]

    row = pl.BlockSpec((r, d), lambda i: (i, 0))
    in_specs = [row, row, row, row, row, row,
                _full((2 * d, d)), _full((1, d)), _full((d, d))]
    out_specs = [row]
    out_shape = [jax.ShapeDtypeStruct((n, d), jnp.float32)]
    if has_next:
        in_specs += [_full((1, d)), _full((1, d)), _full((2 * d, 2 * d)),
                     _full((1, 2 * d))]
        out_specs += [row, pl.BlockSpec((4, r, d), lambda i: (0, i, 0))]
        out_shape += [jax.ShapeDtypeStruct((n, d), jnp.float32),
                      jax.ShapeDtypeStruct((4, n, d), jnp.float32)]
    return pl.pallas_call(body, grid=(n // r,), in_specs=in_specs,
                          out_specs=out_specs, out_shape=out_shape)


def _sc_mesh_params(n, e):
    info = plsc.get_sparse_core_info()
    ns = info.num_subcores
    CH = 80
    per_tile = e // ns
    nch = per_tile // CH
    assert per_tile % CH == 0 and nch % 2 == 0
    rows_main = (n // ns) // 8 * 8
    rows_tail = n - rows_main * ns
    assert rows_tail % 8 == 0
    return ns, CH, per_tile, nch, rows_main, rows_tail


def _make_sc_segment(n, d, e):
    ns, CH0, per_tile, _, rows_main, rows_tail = _sc_mesh_params(n, e)
    NB = 4
    CH = 40
    nch = per_tile // CH
    assert per_tile % CH == 0 and nch % NB == 0

    mesh = plsc.VectorSubcoreMesh(core_axis_name="c", subcore_axis_name="s")
    out_type = (jax.ShapeDtypeStruct((n, d), jnp.float32),
                jax.ShapeDtypeStruct((n, d), jnp.float32))
    idx = lambda: pltpu.VMEM((CH,), jnp.int32)
    row = lambda: pltpu.VMEM((CH, d), jnp.float32)
    scratch = tuple(
        kind() for _ in range(NB) for kind in (idx, idx, idx, idx, idx,
                                               row, row)
    ) + (pltpu.VMEM_SHARED((n, d), jnp.float32),) \
      + (pltpu.SemaphoreType.DMA,) * (5 * NB)

    def body(T_ref, ed_ref, S0_ref, S1_ref, *rest):
        nscr = 7 * NB
        bufs_flat = rest[:nscr]
        S_sh = rest[nscr]
        sems = rest[nscr + 1:]
        B = []
        for b in range(NB):
            di, si, gA, gB, dc, a, bv = bufs_flat[7 * b:7 * b + 7]
            sI, sJ, sA, sB_, sS = sems[5 * b:5 * b + 5]
            B.append({"di": di, "si": si, "gA": gA, "gB": gB, "dc": dc,
                      "a": a, "b": bv, "sI": sI, "sJ": sJ, "sA": sA,
                      "sB": sB_, "sS": sS})
        c = lax.axis_index("c")
        s = lax.axis_index("s")
        base = s * per_tile
        offA = c * n
        offB = (2 + c) * n
        a0 = B[0]["a"]

        @pl.loop(0, CH)
        def _(rr):
            for j in range(d // 16):
                a0[rr, pl.ds(j * 16, 16)] = jnp.zeros((16,), jnp.float32)

        nz = rows_main // CH
        rem = rows_main - nz * CH
        for k in range(nz):
            rs = pl.ds(s * rows_main + k * CH, CH)
            pltpu.sync_copy(a0, S_sh.at[rs])
        if rem:
            rs = pl.ds(s * rows_main + nz * CH, rem)
            pltpu.sync_copy(a0.at[pl.ds(0, rem)], S_sh.at[rs])
        if rows_tail:
            @pl.when(s == 0)
            def _():
                rt = pl.ds(rows_main * ns, rows_tail)
                pltpu.sync_copy(a0.at[pl.ds(0, rows_tail)], S_sh.at[rt])
        plsc.subcore_barrier()

        def issue_idx(q, u):
            off = base + q * CH
            pltpu.async_copy(ed_ref.at[pl.ds(off, CH)], u["di"], u["sI"])
            pltpu.async_copy(ed_ref.at[pl.ds(e + off, CH)], u["si"], u["sJ"])

        def wait_idx(u):
            pltpu.make_async_copy(ed_ref.at[pl.ds(0, CH)], u["di"],
                                  u["sI"]).wait()
            pltpu.make_async_copy(ed_ref.at[pl.ds(0, CH)], u["si"],
                                  u["sJ"]).wait()

        offs16 = list(range(0, CH - 16, 16)) + [CH - 16]

        def adjust_and_gather(u):
            for o in offs16:
                sl = pl.ds(o, 16)
                u["gA"][sl] = u["di"][sl] + offA
                u["gB"][sl] = u["si"][sl] + offB
            pltpu.async_copy(T_ref.at[u["gA"]], u["a"], u["sA"])
            pltpu.async_copy(T_ref.at[u["gB"]], u["b"], u["sB"])

        def wait_gather(u):
            pltpu.make_async_copy(T_ref.at[u["gA"]], u["a"], u["sA"]).wait()
            pltpu.make_async_copy(T_ref.at[u["gB"]], u["b"], u["sB"]).wait()

        def compute(u):
            for o in offs16:
                sl = pl.ds(o, 16)
                u["dc"][sl] = u["gA"][sl] - offA
            aV, bV = u["a"], u["b"]

            @pl.loop(0, CH)
            def _(rr):
                for j in range(d // 16):
                    sl = pl.ds(j * 16, 16)
                    aV[rr, sl] = jnp.maximum(aV[rr, sl] + bV[rr, sl], 0.0)

        def issue_scatter(u):
            pltpu.async_copy(u["a"], S_sh.at[u["dc"]], u["sS"], add=True)

        def drain_scatter(u):
            pltpu.make_async_copy(u["a"], S_sh.at[u["dc"]], u["sS"]).wait()

        for b in range(NB):
            issue_idx(b, B[b])
            wait_idx(B[b])
            adjust_and_gather(B[b])
            issue_idx(NB + b, B[b])

        @pl.loop(0, nch // NB)
        def _(i):
            for b in range(NB):
                u = B[b]
                wait_gather(u)
                compute(u)
                issue_scatter(u)
                pb = B[(b - 1) % NB]
                qq = i * NB + b - 1

                @pl.when((qq >= 0) & (qq + NB < nch))
                def _():
                    wait_idx(pb)
                    drain_scatter(pb)
                    adjust_and_gather(pb)

                    @pl.when(qq + 2 * NB < nch)
                    def _():
                        issue_idx(qq + 2 * NB, pb)

        for b in range(NB):
            drain_scatter(B[b])
        plsc.subcore_barrier()

        def copy_rows(src, dst):
            rs = pl.ds(s * rows_main, rows_main)
            pltpu.sync_copy(src.at[rs], dst.at[rs])
            if rows_tail:
                @pl.when(s == 0)
                def _():
                    rt = pl.ds(rows_main * ns, rows_tail)
                    pltpu.sync_copy(src.at[rt], dst.at[rt])

        @pl.when(c == 0)
        def _():
            copy_rows(S_sh, S0_ref)

        @pl.when(c == 1)
        def _():
            copy_rows(S_sh, S1_ref)

    return pl.kernel(body, out_type=out_type, mesh=mesh,
                     scratch_types=scratch)


def _make_sc_count(n, d, e):
    ns, CH, per_tile, nch, rows_main, rows_tail = _sc_mesh_params(n, e)
    per_w = e // (2 * ns)
    nchw = per_w // CH
    assert per_w % CH == 0
    pair = nchw // 2
    mesh = plsc.VectorSubcoreMesh(core_axis_name="c", subcore_axis_name="s")
    idx = lambda: pltpu.VMEM((CH,), jnp.int32)

    def body(ed_ref, ones_ref, cnt0_ref, cnt1_ref,
             di0, di1, dc0, dc1, ones_v, zv, cnt_sh, sI0, sI1, sS0, sS1):
        c = lax.axis_index("c")
        s = lax.axis_index("s")
        base = (c * ns + s) * per_w
        B = ({"di": di0, "dc": dc0, "sI": sI0, "sS": sS0},
             {"di": di1, "dc": dc1, "sI": sI1, "sS": sS1})

        def copy_rows(src, dst):
            rs = pl.ds(s * rows_main, rows_main)
            pltpu.sync_copy(src.at[rs], dst.at[rs])
            if rows_tail:
                @pl.when(s == 0)
                def _():
                    rt = pl.ds(rows_main * ns, rows_tail)
                    pltpu.sync_copy(src.at[rt], dst.at[rt])

        @pl.loop(0, CH)
        def _(rr):
            for j in range(d // 16):
                zv[rr, pl.ds(j * 16, 16)] = jnp.zeros((16,), jnp.float32)

        nz = rows_main // CH
        rem = rows_main - nz * CH
        for k in range(nz):
            rs = pl.ds(s * rows_main + k * CH, CH)
            pltpu.sync_copy(zv, cnt_sh.at[rs])
        if rem:
            rs = pl.ds(s * rows_main + nz * CH, rem)
            pltpu.sync_copy(zv.at[pl.ds(0, rem)], cnt_sh.at[rs])
        if rows_tail:
            @pl.when(s == 0)
            def _():
                rt = pl.ds(rows_main * ns, rows_tail)
                pltpu.sync_copy(zv.at[pl.ds(0, rows_tail)], cnt_sh.at[rt])
        pltpu.sync_copy(ones_ref, ones_v)
        plsc.subcore_barrier()

        def issue_idx(q, u):
            pltpu.async_copy(ed_ref.at[pl.ds(base + q * CH, CH)],
                             u["di"], u["sI"])

        def wait_idx(u):
            pltpu.make_async_copy(ed_ref.at[pl.ds(0, CH)], u["di"],
                                  u["sI"]).wait()

        issue_idx(0, B[0])
        issue_idx(1, B[1])

        @pl.loop(0, pair)
        def _(i):
            q = i * 2
            for b in range(2):
                u = B[b]
                wait_idx(u)

                @pl.when(i > 0)
                def _():
                    pltpu.make_async_copy(ones_v, cnt_sh.at[u["dc"]],
                                          u["sS"]).wait()

                for j in range(CH // 16):
                    sl = pl.ds(j * 16, 16)
                    u["dc"][sl] = u["di"][sl]

                @pl.when(q + 2 + b < nchw)
                def _():
                    issue_idx(q + 2 + b, u)

                pltpu.async_copy(ones_v, cnt_sh.at[u["dc"]], u["sS"],
                                 add=True)

        if nchw % 2:
            u = B[0]
            wait_idx(u)
            pltpu.make_async_copy(ones_v, cnt_sh.at[u["dc"]], u["sS"]).wait()
            for j in range(CH // 16):
                sl = pl.ds(j * 16, 16)
                u["dc"][sl] = u["di"][sl]
            pltpu.async_copy(ones_v, cnt_sh.at[u["dc"]], u["sS"], add=True)

        pltpu.make_async_copy(ones_v, cnt_sh.at[B[0]["dc"]], B[0]["sS"]).wait()
        pltpu.make_async_copy(ones_v, cnt_sh.at[B[1]["dc"]], B[1]["sS"]).wait()
        plsc.subcore_barrier()

        @pl.when(c == 0)
        def _():
            copy_rows(cnt_sh, cnt0_ref)

        @pl.when(c == 1)
        def _():
            copy_rows(cnt_sh, cnt1_ref)

    return pl.kernel(
        body,
        out_type=(jax.ShapeDtypeStruct((n, d), jnp.float32),
                  jax.ShapeDtypeStruct((n, d), jnp.float32)),
        mesh=mesh,
        scratch_types=(idx(), idx(), idx(), idx(),
                       pltpu.VMEM((CH, d), jnp.float32),
                       pltpu.VMEM((CH, d), jnp.float32),
                       pltpu.VMEM_SHARED((n, d), jnp.float32))
                      + (pltpu.SemaphoreType.DMA,) * 4)


def kernel(x, edge_index, enc_W, enc_b, enc_ln_g, enc_ln_b, ln_g, ln_b,
           W1, b1, W2, b2, Wr):
    n, d = x.shape
    e = edge_index.shape[1]
    nlayers = ln_g.shape[0]
    r = 1000

    src = edge_index[0].astype(jnp.int32)
    dst = edge_index[1].astype(jnp.int32)
    edges2 = jnp.concatenate([dst, src])
    row2 = lambda v: v.reshape(1, -1).astype(jnp.float32)

    encode = _make_tc_encode(n, d, r)
    xcur, hh, T = encode(x, enc_W, row2(enc_b), row2(enc_ln_g),
                         row2(enc_ln_b), row2(ln_g[0]), row2(ln_b[0]),
                         W1[0], row2(b1[0]))

    sc_seg = _make_sc_segment(n, d, e)
    ones_rows = jnp.ones((80, d), jnp.float32)
    cnt0, cnt1 = _make_sc_count(n, d, e)(edges2, ones_rows)

    for l in range(nlayers):
        S0, S1 = sc_seg(T.reshape(4 * n, d), edges2)
        W2p = W2[l]
        has_next = l + 1 < nlayers
        update = _make_tc_update(n, d, r, has_next)
        if has_next:
            xcur, hh, T = update(xcur, hh, S0, S1, cnt0, cnt1, W2p,
                                 row2(b2[l]), Wr[l], row2(ln_g[l + 1]),
                                 row2(ln_b[l + 1]), W1[l + 1],
                                 row2(b1[l + 1]))
        else:
            (xcur,) = update(xcur, hh, S0, S1, cnt0, cnt1, W2p,
                             row2(b2[l]), Wr[l])
    return xcur


# speedup vs baseline: 2.5046x; 1.0055x over previous
import functools

import jax
import jax.numpy as jnp
import numpy as np
from jax import lax
from jax.experimental import pallas as pl
from jax.experimental.pallas import tpu as pltpu
from jax.experimental.pallas import tpu_sc as plsc


def _ln(h, g, b):
    mu = jnp.mean(h, axis=-1, keepdims=True)
    var = jnp.mean((h - mu) ** 2, axis=-1, keepdims=True)
    return (h - mu) / jnp.sqrt(var + 1e-5) * g + b


def _dot(a, b):
    return jnp.dot(a, b, preferred_element_type=jnp.float32)


def _full(shape):
    return pl.BlockSpec(shape, lambda i: (0,) * len(shape))


def _make_tc_encode(n, d, r):
    def body(x_ref, encW_ref, encb_ref, encg_ref, encbb_ref, g0_ref, b0_ref,
             W1_ref, b1_ref, xcur_ref, hh_ref, T_ref):
        h = _dot(x_ref[...], encW_ref[...]) + encb_ref[...]
        h = _ln(h, encg_ref[...], encbb_ref[...])
        xcur_ref[...] = h
        hh = jnp.maximum(_ln(h, g0_ref[...], b0_ref[...]), 0.0)
        hh_ref[...] = hh
        W1 = W1_ref[...]
        A = _dot(hh, W1[:d, :]) + b1_ref[...]
        B = _dot(hh, W1[d:, :])
        T_ref[0] = A[:, :d]
        T_ref[1] = A[:, d:]
        T_ref[2] = B[:, :d]
        T_ref[3] = B[:, d:]

    row = pl.BlockSpec((r, d), lambda i: (i, 0))
    return pl.pallas_call(
        body,
        grid=(n // r,),
        in_specs=[row, _full((d, d)), _full((1, d)), _full((1, d)),
                  _full((1, d)), _full((1, d)), _full((1, d)),
                  _full((2 * d, 2 * d)), _full((1, 2 * d))],
        out_specs=[row, row, pl.BlockSpec((4, r, d), lambda i: (0, i, 0))],
        out_shape=[jax.ShapeDtypeStruct((n, d), jnp.float32),
                   jax.ShapeDtypeStruct((n, d), jnp.float32),
                   jax.ShapeDtypeStruct((4, n, d), jnp.float32)],
    )


def _make_tc_update(n, d, r, has_next):
    def body(xcur_ref, hh_ref, S0_ref, S1_ref, cnt0_ref, cnt1_ref, W2_ref,
             b2_ref, Wr_ref, *rest):
        if has_next:
            gn_ref, bn_ref, W1n_ref, b1n_ref = rest[:4]
            xnew_ref, hh2_ref, T_ref = rest[4:]
        else:
            (xnew_ref,) = rest
        cc = cnt0_ref[...][:, 0:1] + cnt1_ref[...][:, 0:1]
        W2 = W2_ref[...]
        m = _dot(S0_ref[...], W2[:d, :]) + _dot(S1_ref[...], W2[d:, :])
        mean = m / jnp.maximum(cc, 1.0) + jnp.minimum(cc, 1.0) * b2_ref[...]
        out = mean + _dot(hh_ref[...], Wr_ref[...])
        xnew = xcur_ref[...] + out
        xnew_ref[...] = xnew
        if has_next:
            hh2 = jnp.maximum(_ln(xnew, gn_ref[...], bn_ref[...]), 0.0)
            hh2_ref[...] = hh2
            W1 = W1n_ref[...]
            A = _dot(hh2, W1[:d, :]) + b1n_ref[...]
            B = _dot(hh2, W1[d:, :])
            T_ref[0] = A[:, :d]
            T_ref[1] = A[:, d:]
            T_ref[2] = B[:, :d]
            T_ref[3] = B[:, d:]

    row = pl.BlockSpec((r, d), lambda i: (i, 0))
    in_specs = [row, row, row, row, row, row,
                _full((2 * d, d)), _full((1, d)), _full((d, d))]
    out_specs = [row]
    out_shape = [jax.ShapeDtypeStruct((n, d), jnp.float32)]
    if has_next:
        in_specs += [_full((1, d)), _full((1, d)), _full((2 * d, 2 * d)),
                     _full((1, 2 * d))]
        out_specs += [row, pl.BlockSpec((4, r, d), lambda i: (0, i, 0))]
        out_shape += [jax.ShapeDtypeStruct((n, d), jnp.float32),
                      jax.ShapeDtypeStruct((4, n, d), jnp.float32)]
    return pl.pallas_call(body, grid=(n // r,), in_specs=in_specs,
                          out_specs=out_specs, out_shape=out_shape)


def _sc_mesh_params(n, e):
    info = plsc.get_sparse_core_info()
    ns = info.num_subcores
    CH = 80
    per_tile = e // ns
    nch = per_tile // CH
    assert per_tile % CH == 0 and nch % 2 == 0
    rows_main = (n // ns) // 8 * 8
    rows_tail = n - rows_main * ns
    assert rows_tail % 8 == 0
    return ns, CH, per_tile, nch, rows_main, rows_tail


def _make_sc_segment(n, d, e):
    ns, CH0, per_tile, _, rows_main, rows_tail = _sc_mesh_params(n, e)
    NB = 4
    CH = 40
    nch = per_tile // CH
    assert per_tile % CH == 0 and nch % NB == 0

    mesh = plsc.VectorSubcoreMesh(core_axis_name="c", subcore_axis_name="s")
    out_type = (jax.ShapeDtypeStruct((n, d), jnp.float32),
                jax.ShapeDtypeStruct((n, d), jnp.float32))
    idx = lambda: pltpu.VMEM((CH,), jnp.int32)
    row = lambda: pltpu.VMEM((CH, d), jnp.float32)
    scratch = tuple(
        kind() for _ in range(NB) for kind in (idx, idx, idx, idx, idx,
                                               row, row)
    ) + (pltpu.VMEM_SHARED((n, d), jnp.float32),) \
      + (pltpu.SemaphoreType.DMA,) * (5 * NB)

    def body(T_ref, ed_ref, S0_ref, S1_ref, *rest):
        nscr = 7 * NB
        bufs_flat = rest[:nscr]
        S_sh = rest[nscr]
        sems = rest[nscr + 1:]
        B = []
        for b in range(NB):
            di, si, gA, gB, dc, a, bv = bufs_flat[7 * b:7 * b + 7]
            sI, sJ, sA, sB_, sS = sems[5 * b:5 * b + 5]
            B.append({"di": di, "si": si, "gA": gA, "gB": gB, "dc": dc,
                      "a": a, "b": bv, "sI": sI, "sJ": sJ, "sA": sA,
                      "sB": sB_, "sS": sS})
        c = lax.axis_index("c")
        s = lax.axis_index("s")
        base = s * per_tile
        offA = c * n
        offB = (2 + c) * n
        a0 = B[0]["a"]

        @pl.loop(0, CH)
        def _(rr):
            for j in range(d // 16):
                a0[rr, pl.ds(j * 16, 16)] = jnp.zeros((16,), jnp.float32)

        nz = rows_main // CH
        rem = rows_main - nz * CH
        for k in range(nz):
            rs = pl.ds(s * rows_main + k * CH, CH)
            pltpu.sync_copy(a0, S_sh.at[rs])
        if rem:
            rs = pl.ds(s * rows_main + nz * CH, rem)
            pltpu.sync_copy(a0.at[pl.ds(0, rem)], S_sh.at[rs])
        if rows_tail:
            @pl.when(s == 0)
            def _():
                rt = pl.ds(rows_main * ns, rows_tail)
                pltpu.sync_copy(a0.at[pl.ds(0, rows_tail)], S_sh.at[rt])
        plsc.subcore_barrier()

        def issue_idx(q, u):
            off = base + q * CH
            pltpu.async_copy(ed_ref.at[pl.ds(off, CH)], u["di"], u["sI"])
            pltpu.async_copy(ed_ref.at[pl.ds(e + off, CH)], u["si"], u["sJ"])

        def wait_idx(u):
            pltpu.make_async_copy(ed_ref.at[pl.ds(0, CH)], u["di"],
                                  u["sI"]).wait()
            pltpu.make_async_copy(ed_ref.at[pl.ds(0, CH)], u["si"],
                                  u["sJ"]).wait()

        offs16 = list(range(0, CH - 16, 16)) + [CH - 16]

        def adjust_and_gather(u):
            for o in offs16:
                sl = pl.ds(o, 16)
                u["gA"][sl] = u["di"][sl] + offA
                u["gB"][sl] = u["si"][sl] + offB
            pltpu.async_copy(T_ref.at[u["gA"]], u["a"], u["sA"])
            pltpu.async_copy(T_ref.at[u["gB"]], u["b"], u["sB"])

        def wait_gather(u):
            pltpu.make_async_copy(T_ref.at[u["gA"]], u["a"], u["sA"]).wait()
            pltpu.make_async_copy(T_ref.at[u["gB"]], u["b"], u["sB"]).wait()

        def compute(u):
            for o in offs16:
                sl = pl.ds(o, 16)
                u["dc"][sl] = u["gA"][sl] - offA
            aV, bV = u["a"], u["b"]

            @pl.loop(0, CH // 2)
            def _(rh):
                rr = rh * 2
                for k in range(2):
                    for j in range(d // 16):
                        sl = pl.ds(j * 16, 16)
                        aV[rr + k, sl] = jnp.maximum(
                            aV[rr + k, sl] + bV[rr + k, sl], 0.0)

        def issue_scatter(u):
            pltpu.async_copy(u["a"], S_sh.at[u["dc"]], u["sS"], add=True)

        def drain_scatter(u):
            pltpu.make_async_copy(u["a"], S_sh.at[u["dc"]], u["sS"]).wait()

        for b in range(NB):
            issue_idx(b, B[b])
            wait_idx(B[b])
            adjust_and_gather(B[b])
            issue_idx(NB + b, B[b])

        @pl.loop(0, nch // NB)
        def _(i):
            for b in range(NB):
                u = B[b]
                wait_gather(u)
                compute(u)
                issue_scatter(u)
                pb = B[(b - 1) % NB]
                qq = i * NB + b - 1

                @pl.when((qq >= 0) & (qq + NB < nch))
                def _():
                    wait_idx(pb)
                    drain_scatter(pb)
                    adjust_and_gather(pb)

                    @pl.when(qq + 2 * NB < nch)
                    def _():
                        issue_idx(qq + 2 * NB, pb)

        for b in range(NB):
            drain_scatter(B[b])
        plsc.subcore_barrier()

        def copy_rows(src, dst):
            rs = pl.ds(s * rows_main, rows_main)
            pltpu.sync_copy(src.at[rs], dst.at[rs])
            if rows_tail:
                @pl.when(s == 0)
                def _():
                    rt = pl.ds(rows_main * ns, rows_tail)
                    pltpu.sync_copy(src.at[rt], dst.at[rt])

        @pl.when(c == 0)
        def _():
            copy_rows(S_sh, S0_ref)

        @pl.when(c == 1)
        def _():
            copy_rows(S_sh, S1_ref)

    return pl.kernel(body, out_type=out_type, mesh=mesh,
                     scratch_types=scratch)


def _make_sc_count(n, d, e):
    ns, CH, per_tile, nch, rows_main, rows_tail = _sc_mesh_params(n, e)
    per_w = e // (2 * ns)
    nchw = per_w // CH
    assert per_w % CH == 0
    pair = nchw // 2
    mesh = plsc.VectorSubcoreMesh(core_axis_name="c", subcore_axis_name="s")
    idx = lambda: pltpu.VMEM((CH,), jnp.int32)

    def body(ed_ref, ones_ref, cnt0_ref, cnt1_ref,
             di0, di1, dc0, dc1, ones_v, zv, cnt_sh, sI0, sI1, sS0, sS1):
        c = lax.axis_index("c")
        s = lax.axis_index("s")
        base = (c * ns + s) * per_w
        B = ({"di": di0, "dc": dc0, "sI": sI0, "sS": sS0},
             {"di": di1, "dc": dc1, "sI": sI1, "sS": sS1})

        def copy_rows(src, dst):
            rs = pl.ds(s * rows_main, rows_main)
            pltpu.sync_copy(src.at[rs], dst.at[rs])
            if rows_tail:
                @pl.when(s == 0)
                def _():
                    rt = pl.ds(rows_main * ns, rows_tail)
                    pltpu.sync_copy(src.at[rt], dst.at[rt])

        @pl.loop(0, CH)
        def _(rr):
            for j in range(d // 16):
                zv[rr, pl.ds(j * 16, 16)] = jnp.zeros((16,), jnp.float32)

        nz = rows_main // CH
        rem = rows_main - nz * CH
        for k in range(nz):
            rs = pl.ds(s * rows_main + k * CH, CH)
            pltpu.sync_copy(zv, cnt_sh.at[rs])
        if rem:
            rs = pl.ds(s * rows_main + nz * CH, rem)
            pltpu.sync_copy(zv.at[pl.ds(0, rem)], cnt_sh.at[rs])
        if rows_tail:
            @pl.when(s == 0)
            def _():
                rt = pl.ds(rows_main * ns, rows_tail)
                pltpu.sync_copy(zv.at[pl.ds(0, rows_tail)], cnt_sh.at[rt])
        pltpu.sync_copy(ones_ref, ones_v)
        plsc.subcore_barrier()

        def issue_idx(q, u):
            pltpu.async_copy(ed_ref.at[pl.ds(base + q * CH, CH)],
                             u["di"], u["sI"])

        def wait_idx(u):
            pltpu.make_async_copy(ed_ref.at[pl.ds(0, CH)], u["di"],
                                  u["sI"]).wait()

        issue_idx(0, B[0])
        issue_idx(1, B[1])

        @pl.loop(0, pair)
        def _(i):
            q = i * 2
            for b in range(2):
                u = B[b]
                wait_idx(u)

                @pl.when(i > 0)
                def _():
                    pltpu.make_async_copy(ones_v, cnt_sh.at[u["dc"]],
                                          u["sS"]).wait()

                for j in range(CH // 16):
                    sl = pl.ds(j * 16, 16)
                    u["dc"][sl] = u["di"][sl]

                @pl.when(q + 2 + b < nchw)
                def _():
                    issue_idx(q + 2 + b, u)

                pltpu.async_copy(ones_v, cnt_sh.at[u["dc"]], u["sS"],
                                 add=True)

        if nchw % 2:
            u = B[0]
            wait_idx(u)
            pltpu.make_async_copy(ones_v, cnt_sh.at[u["dc"]], u["sS"]).wait()
            for j in range(CH // 16):
                sl = pl.ds(j * 16, 16)
                u["dc"][sl] = u["di"][sl]
            pltpu.async_copy(ones_v, cnt_sh.at[u["dc"]], u["sS"], add=True)

        pltpu.make_async_copy(ones_v, cnt_sh.at[B[0]["dc"]], B[0]["sS"]).wait()
        pltpu.make_async_copy(ones_v, cnt_sh.at[B[1]["dc"]], B[1]["sS"]).wait()
        plsc.subcore_barrier()

        @pl.when(c == 0)
        def _():
            copy_rows(cnt_sh, cnt0_ref)

        @pl.when(c == 1)
        def _():
            copy_rows(cnt_sh, cnt1_ref)

    return pl.kernel(
        body,
        out_type=(jax.ShapeDtypeStruct((n, d), jnp.float32),
                  jax.ShapeDtypeStruct((n, d), jnp.float32)),
        mesh=mesh,
        scratch_types=(idx(), idx(), idx(), idx(),
                       pltpu.VMEM((CH, d), jnp.float32),
                       pltpu.VMEM((CH, d), jnp.float32),
                       pltpu.VMEM_SHARED((n, d), jnp.float32))
                      + (pltpu.SemaphoreType.DMA,) * 4)


def kernel(x, edge_index, enc_W, enc_b, enc_ln_g, enc_ln_b, ln_g, ln_b,
           W1, b1, W2, b2, Wr):
    n, d = x.shape
    e = edge_index.shape[1]
    nlayers = ln_g.shape[0]
    r = 1000

    src = edge_index[0].astype(jnp.int32)
    dst = edge_index[1].astype(jnp.int32)
    edges2 = jnp.concatenate([dst, src])
    row2 = lambda v: v.reshape(1, -1).astype(jnp.float32)

    encode = _make_tc_encode(n, d, r)
    xcur, hh, T = encode(x, enc_W, row2(enc_b), row2(enc_ln_g),
                         row2(enc_ln_b), row2(ln_g[0]), row2(ln_b[0]),
                         W1[0], row2(b1[0]))

    sc_seg = _make_sc_segment(n, d, e)
    ones_rows = jnp.ones((80, d), jnp.float32)
    cnt0, cnt1 = _make_sc_count(n, d, e)(edges2, ones_rows)

    for l in range(nlayers):
        S0, S1 = sc_seg(T.reshape(4 * n, d), edges2)
        W2p = W2[l]
        has_next = l + 1 < nlayers
        update = _make_tc_update(n, d, r, has_next)
        if has_next:
            xcur, hh, T = update(xcur, hh, S0, S1, cnt0, cnt1, W2p,
                                 row2(b2[l]), Wr[l], row2(ln_g[l + 1]),
                                 row2(ln_b[l + 1]), W1[l + 1],
                                 row2(b1[l + 1]))
        else:
            (xcur,) = update(xcur, hh, S0, S1, cnt0, cnt1, W2p,
                             row2(b2[l]), Wr[l])
    return xcur
